# baseline (speedup 1.0000x reference)
import functools

import jax
import jax.numpy as jnp
from jax import lax
from jax.experimental import pallas as pl
from jax.experimental.pallas import tpu as pltpu
from jax.experimental.pallas import tpu_sc as plsc

BATCH = 16384
NUM_FIELDS = 26
VOCAB = 1000000
LANES = 16
NUM_CORES = 2
NUM_SUBCORES = 16
NUM_WORKERS = NUM_CORES * NUM_SUBCORES
BPW = BATCH // NUM_WORKERS
CHUNK = 128
CPF = BPW // CHUNK


def _lr_body(idx_hbm, w_hbm, b_hbm, out_hbm, idx_v, rows_v, acc_v, bias_v, sem):
    wid = lax.axis_index("s") * NUM_CORES + lax.axis_index("c")
    base = wid * BPW

    pltpu.sync_copy(b_hbm, bias_v)
    pltpu.sync_copy(idx_hbm.at[:, pl.ds(base, BPW)], idx_v)

    def _off_field(f, _):
        def _off_vec(j, _):
            sl = pl.ds(j * LANES, LANES)
            idx_v[f, sl] = idx_v[f, sl] + f * VOCAB
            return 0
        return lax.fori_loop(0, BPW // LANES, _off_vec, 0)

    lax.fori_loop(0, NUM_FIELDS, _off_field, 0)

    def _gather(c, _):
        f = c // CPF
        j = c % CPF
        sl = pl.ds(j * CHUNK, CHUNK)
        pltpu.async_copy(w_hbm.at[idx_v.at[f, sl]], rows_v.at[f, sl], sem).wait()
        return 0

    lax.fori_loop(0, NUM_FIELDS * CPF, _gather, 0)

    bvec = bias_v[...]

    def _reduce(v, _):
        sl = pl.ds(v * LANES, LANES)

        def _acc_field(f, s):
            return s + rows_v[f, sl]

        acc_v[sl] = lax.fori_loop(0, NUM_FIELDS, _acc_field, bvec)
        return 0

    lax.fori_loop(0, BPW // LANES, _reduce, 0)

    pltpu.sync_copy(acc_v, out_hbm.at[pl.ds(base, BPW)])


@jax.jit
def _lr_call(idx_t, w_flat, bias16):
    mesh = plsc.VectorSubcoreMesh(
        core_axis_name="c", subcore_axis_name="s",
        num_cores=NUM_CORES, num_subcores=NUM_SUBCORES,
    )
    return pl.kernel(
        _lr_body,
        out_type=jax.ShapeDtypeStruct((BATCH,), jnp.float32),
        mesh=mesh,
        scratch_types=[
            pltpu.VMEM((NUM_FIELDS, BPW), jnp.int32),
            pltpu.VMEM((NUM_FIELDS, BPW), jnp.float32),
            pltpu.VMEM((BPW,), jnp.float32),
            pltpu.VMEM((LANES,), jnp.float32),
            pltpu.SemaphoreType.DMA,
        ],
    )(idx_t, w_flat, bias16)


def kernel(indices, W, bias):
    idx_t = indices.astype(jnp.int32).T
    w_flat = W.reshape(NUM_FIELDS * VOCAB)
    bias16 = jnp.broadcast_to(bias.astype(jnp.float32), (LANES,))
    out = _lr_call(idx_t, w_flat, bias16)
    return out.reshape(BATCH, 1)


# speedup vs baseline: 1.0270x; 1.0270x over previous
import functools

import jax
import jax.numpy as jnp
from jax import lax
from jax.experimental import pallas as pl
from jax.experimental.pallas import tpu as pltpu
from jax.experimental.pallas import tpu_sc as plsc

BATCH = 16384
NUM_FIELDS = 26
VOCAB = 1000000
LANES = 16
NUM_CORES = 2
NUM_SUBCORES = 16
NUM_WORKERS = NUM_CORES * NUM_SUBCORES
BPW = BATCH // NUM_WORKERS
VPF = BPW // LANES
FLAT = NUM_FIELDS * BPW


def _lr_body(idx_hbm, w_hbm, b_hbm, out_hbm, idx2_v, flat_v, rows_v, acc_v,
             bias_v, sem):
    wid = lax.axis_index("s") * NUM_CORES + lax.axis_index("c")
    base = wid * BPW

    pltpu.sync_copy(b_hbm, bias_v)
    pltpu.sync_copy(idx_hbm.at[:, pl.ds(base, BPW)], idx2_v)

    def _off_field(f, _):
        for j in range(VPF):
            dst = pl.ds(f * BPW + j * LANES, LANES)
            flat_v[dst] = idx2_v[f, pl.ds(j * LANES, LANES)] + f * VOCAB
        return 0

    lax.fori_loop(0, NUM_FIELDS, _off_field, 0)

    pltpu.async_copy(w_hbm.at[flat_v], rows_v, sem).wait()

    bvec = bias_v[...]

    def _reduce(v, _):
        s = bvec
        for f in range(NUM_FIELDS):
            s = s + rows_v[pl.ds(f * BPW + v * LANES, LANES)]
        acc_v[pl.ds(v * LANES, LANES)] = s
        return 0

    lax.fori_loop(0, VPF, _reduce, 0)

    pltpu.sync_copy(acc_v, out_hbm.at[pl.ds(base, BPW)])


@jax.jit
def _lr_call(idx_t, w_flat, bias16):
    mesh = plsc.VectorSubcoreMesh(
        core_axis_name="c", subcore_axis_name="s",
        num_cores=NUM_CORES, num_subcores=NUM_SUBCORES,
    )
    return pl.kernel(
        _lr_body,
        out_type=jax.ShapeDtypeStruct((BATCH,), jnp.float32),
        mesh=mesh,
        scratch_types=[
            pltpu.VMEM((NUM_FIELDS, BPW), jnp.int32),
            pltpu.VMEM((FLAT,), jnp.int32),
            pltpu.VMEM((FLAT,), jnp.float32),
            pltpu.VMEM((BPW,), jnp.float32),
            pltpu.VMEM((LANES,), jnp.float32),
            pltpu.SemaphoreType.DMA,
        ],
    )(idx_t, w_flat, bias16)


def kernel(indices, W, bias):
    idx_t = indices.astype(jnp.int32).T
    w_flat = W.reshape(NUM_FIELDS * VOCAB)
    bias16 = jnp.broadcast_to(bias.astype(jnp.float32), (LANES,))
    out = _lr_call(idx_t, w_flat, bias16)
    return out.reshape(BATCH, 1)
